# baseline (speedup 1.0000x reference)
import functools

import jax
import jax.numpy as jnp
from jax import lax
from jax.experimental import pallas as pl
from jax.experimental.pallas import tpu as pltpu
from jax.experimental.pallas import tpu_sc as plsc

VOCAB = 1000000
D = 32
NC, NS = 2, 16
NW = NC * NS
B_TOT = 16384 * 50
B_PER_W = B_TOT // NW
CHUNK = 128
NCHUNK = B_PER_W // CHUNK
K = 10
G = NCHUNK // K

_ROW_BYTES = CHUNK * D * 4


def _emb_body(table_hbm, idx_hbm, out_hbm, idx_v, rows_v, gsem_a, gsem_b,
              wsem_a, wsem_b):
    wid = lax.axis_index("s") * NC + lax.axis_index("c")
    pltpu.sync_copy(idx_hbm.at[wid], idx_v)

    def fire_gathers(g, half, sem):
        for b in range(K):
            j = g * K + b
            pltpu.async_copy(table_hbm.at[idx_v.at[j]],
                             rows_v.at[half * K + b], sem)

    def drain_gathers(sem):
        for _ in range(K):
            pltpu.make_async_copy(table_hbm.at[idx_v.at[0]],
                                  rows_v.at[0], sem).wait()

    row0 = wid * B_PER_W

    def fire_writes(g, half, sem):
        for b in range(K):
            j = g * K + b
            pltpu.async_copy(rows_v.at[half * K + b],
                             out_hbm.at[pl.ds(row0 + j * CHUNK, CHUNK)], sem)

    def drain_writes(sem):
        for _ in range(K):
            pltpu.make_async_copy(rows_v.at[0],
                                  out_hbm.at[pl.ds(0, CHUNK)], sem).wait()

    fire_gathers(0, 0, gsem_a)

    def body(i, carry):
        g = i * 2
        fire_gathers(g + 1, 1, gsem_b)
        drain_gathers(gsem_a)
        fire_writes(g, 0, wsem_a)
        drain_writes(wsem_a)
        @pl.when(g + 2 < G)
        def _():
            fire_gathers(g + 2, 0, gsem_a)
        drain_gathers(gsem_b)
        fire_writes(g + 1, 1, wsem_b)
        drain_writes(wsem_b)
        return carry

    lax.fori_loop(0, G // 2, body, 0)


@functools.partial(
    pl.kernel,
    out_type=jax.ShapeDtypeStruct((B_TOT, D), jnp.float32),
    mesh=plsc.VectorSubcoreMesh(core_axis_name="c", subcore_axis_name="s"),
    scratch_types=[
        pltpu.VMEM((NCHUNK, CHUNK), jnp.int32),
        pltpu.VMEM((2 * K, CHUNK, D), jnp.float32),
        pltpu.SemaphoreType.DMA,
        pltpu.SemaphoreType.DMA,
        pltpu.SemaphoreType.DMA,
        pltpu.SemaphoreType.DMA,
    ],
    compiler_params=pltpu.CompilerParams(use_tc_tiling_on_sc=False),
)
def _emb_lookup(table_hbm, idx_hbm, out_hbm, idx_v, rows_v, gsem_a, gsem_b,
                wsem_a, wsem_b):
    _emb_body(table_hbm, idx_hbm, out_hbm, idx_v, rows_v, gsem_a, gsem_b,
              wsem_a, wsem_b)


_VB = 8192
_TGRID = (VOCAB + _VB - 1) // _VB


def _transpose_body(wt_ref, out_ref):
    out_ref[...] = wt_ref[...].T


_tc_table_transpose = pl.pallas_call(
    _transpose_body,
    grid=(_TGRID,),
    in_specs=[pl.BlockSpec((D, _VB), lambda i: (0, i))],
    out_specs=pl.BlockSpec((_VB, D), lambda i: (i, 0)),
    out_shape=jax.ShapeDtypeStruct((VOCAB, D), jnp.float32),
)


def kernel(input_ids, weight_shard):
    idx = input_ids.astype(jnp.int32).reshape(NW, NCHUNK, CHUNK)
    table_rm = _tc_table_transpose(weight_shard.T)
    out = _emb_lookup(table_rm, idx)
    return out.reshape(input_ids.shape[0], input_ids.shape[1], D)


# speedup vs baseline: 1.4222x; 1.4222x over previous
import functools

import jax
import jax.numpy as jnp
from jax import lax
from jax.experimental import pallas as pl
from jax.experimental.pallas import tpu as pltpu
from jax.experimental.pallas import tpu_sc as plsc

VOCAB = 1000000
D = 32
NC, NS = 2, 16
NW = NC * NS
B = 16384
H = 50
B_TOT = B * H
CHUNK = 128
NBT = B // CHUNK
BT_PER_W = NBT // NW
NCHUNK = H * BT_PER_W
K = 10
G = NCHUNK // K
TB = 2


def _emb_body(table_hbm, ids_hbm, out_hbm, idsv_v, rows_v, tbuf_v,
              gsem_a, gsem_b, wsem, isem):
    wid = lax.axis_index("s") * NC + lax.axis_index("c")
    b0 = wid * BT_PER_W * CHUNK

    pltpu.async_copy(ids_hbm.at[:, pl.ds(b0, BT_PER_W * CHUNK)], idsv_v,
                     isem).wait()

    def chunk_idx(j):
        h = j // BT_PER_W
        q = j - h * BT_PER_W
        return h, q

    def fire_gathers(g, half, sem):
        for bb in range(K):
            j = g * K + bb
            h, q = chunk_idx(j)
            pltpu.async_copy(
                table_hbm.at[idsv_v.at[h, pl.ds(q * CHUNK, CHUNK)]],
                rows_v.at[half * K + bb], sem)

    def drain_gathers(sem):
        for _ in range(K):
            pltpu.make_async_copy(
                table_hbm.at[idsv_v.at[0, pl.ds(0, CHUNK)]],
                rows_v.at[0], sem).wait()

    def transpose_write(g, half):
        def tloop(t, carry):
            @pl.when(t >= TB)
            def _():
                pltpu.make_async_copy(tbuf_v.at[0], out_hbm.at[0, :, 0],
                                      wsem).wait()
            tb = lax.rem(t, TB)
            src = rows_v.at[half * K + t]
            for kk in range(8):
                ridx = lax.iota(jnp.int32, 16) + (kk * 16)
                for d in range(D):
                    col = jnp.full((16,), d, jnp.int32)
                    v = plsc.load_gather(src, [ridx, col])
                    tbuf_v[tb, d // 8, d % 8, pl.ds(kk * 16, 16)] = v
            j = g * K + t
            h, q = chunk_idx(j)
            pltpu.async_copy(tbuf_v.at[tb],
                             out_hbm.at[h, :, wid * BT_PER_W + q], wsem)
            return carry

        lax.fori_loop(0, K, tloop, 0)
        for _ in range(TB):
            pltpu.make_async_copy(tbuf_v.at[0], out_hbm.at[0, :, 0],
                                  wsem).wait()

    fire_gathers(0, 0, gsem_a)

    def body(i, carry):
        g = i * 2
        fire_gathers(g + 1, 1, gsem_b)
        drain_gathers(gsem_a)
        transpose_write(g, 0)
        @pl.when(g + 2 < G)
        def _():
            fire_gathers(g + 2, 0, gsem_a)
        drain_gathers(gsem_b)
        transpose_write(g + 1, 1)
        return carry

    lax.fori_loop(0, G // 2, body, 0)


@functools.partial(
    pl.kernel,
    out_type=jax.ShapeDtypeStruct((H, D // 8, NBT, 8, CHUNK), jnp.float32),
    mesh=plsc.VectorSubcoreMesh(core_axis_name="c", subcore_axis_name="s"),
    scratch_types=[
        pltpu.VMEM((H, BT_PER_W * CHUNK), jnp.int32),
        pltpu.VMEM((2 * K, CHUNK, D), jnp.float32),
        pltpu.VMEM((TB, D // 8, 8, CHUNK), jnp.float32),
        pltpu.SemaphoreType.DMA,
        pltpu.SemaphoreType.DMA,
        pltpu.SemaphoreType.DMA,
        pltpu.SemaphoreType.DMA,
    ],
    compiler_params=pltpu.CompilerParams(use_tc_tiling_on_sc=False,
                                         needs_layout_passes=False),
)
def _emb_lookup(table_hbm, ids_hbm, out_hbm, idsv_v, rows_v, tbuf_v,
                gsem_a, gsem_b, wsem, isem):
    _emb_body(table_hbm, ids_hbm, out_hbm, idsv_v, rows_v, tbuf_v,
              gsem_a, gsem_b, wsem, isem)


def kernel(input_ids, weight_shard):
    ids_t = input_ids.astype(jnp.int32).T
    out5 = _emb_lookup(weight_shard, ids_t)
    return out5.transpose(2, 4, 0, 1, 3).reshape(B, H, D)


# speedup vs baseline: 2.4189x; 1.7009x over previous
import functools

import jax
import jax.numpy as jnp
from jax import lax
from jax.experimental import pallas as pl
from jax.experimental.pallas import tpu as pltpu
from jax.experimental.pallas import tpu_sc as plsc

VOCAB = 1000000
D = 32
NC, NS = 2, 16
NW = NC * NS
B = 16384
H = 50
B_TOT = B * H
CHUNK = 128
NBT = B // CHUNK
BT_PER_W = NBT // NW
NCHUNK = H * BT_PER_W
K = 10
G = NCHUNK // K
TB = 4
SKEW = 129


def _emb_body(table_hbm, ids_hbm, out_hbm, idsv_v, rows_v, tbuf_v,
              gsem_a, gsem_b, wsem, isem):
    wid = lax.axis_index("s") * NC + lax.axis_index("c")
    b0 = wid * BT_PER_W * CHUNK

    pltpu.async_copy(ids_hbm.at[:, pl.ds(b0, BT_PER_W * CHUNK)], idsv_v,
                     isem).wait()

    def chunk_idx(j):
        h = j // BT_PER_W
        q = j - h * BT_PER_W
        return h, q

    def fire_gathers(g, half, sem):
        for bb in range(K):
            j = g * K + bb
            h, q = chunk_idx(j)
            pltpu.async_copy(
                table_hbm.at[idsv_v.at[h, pl.ds(q * CHUNK, CHUNK)]],
                rows_v.at[half * K + bb], sem)

    def drain_gathers(sem):
        for _ in range(K):
            pltpu.make_async_copy(
                table_hbm.at[idsv_v.at[0, pl.ds(0, CHUNK)]],
                rows_v.at[0], sem).wait()

    lo_d = lax.iota(jnp.int32, 16)
    hi_d = lo_d + 16

    def transpose_write(g, half):
        def tloop(t, carry):
            @pl.when(t >= TB)
            def _():
                for _dt in range(D // 8):
                    pltpu.make_async_copy(tbuf_v.at[0, pl.ds(0, 8),
                                                    pl.ds(0, CHUNK)],
                                          out_hbm.at[0, 0, 0], wsem).wait()
            tb = lax.rem(t, TB)
            src = rows_v.at[half * K + t]
            for bl in range(CHUNK):
                col = jnp.full((16,), bl, jnp.int32)
                plsc.store_scatter(tbuf_v.at[tb], [lo_d, col],
                                   src[bl, pl.ds(0, 16)])
                plsc.store_scatter(tbuf_v.at[tb], [hi_d, col],
                                   src[bl, pl.ds(16, 16)])
            j = g * K + t
            h, q = chunk_idx(j)
            for dt in range(D // 8):
                pltpu.async_copy(
                    tbuf_v.at[tb, pl.ds(dt * 8, 8), pl.ds(0, CHUNK)],
                    out_hbm.at[h, dt, wid * BT_PER_W + q], wsem)
            return carry

        lax.fori_loop(0, K, tloop, 0)
        for _ in range(TB * (D // 8)):
            pltpu.make_async_copy(tbuf_v.at[0, pl.ds(0, 8), pl.ds(0, CHUNK)],
                                  out_hbm.at[0, 0, 0], wsem).wait()

    fire_gathers(0, 0, gsem_a)

    def body(i, carry):
        g = i * 2
        fire_gathers(g + 1, 1, gsem_b)
        drain_gathers(gsem_a)
        transpose_write(g, 0)
        @pl.when(g + 2 < G)
        def _():
            fire_gathers(g + 2, 0, gsem_a)
        drain_gathers(gsem_b)
        transpose_write(g + 1, 1)
        return carry

    lax.fori_loop(0, G // 2, body, 0)


@functools.partial(
    pl.kernel,
    out_type=jax.ShapeDtypeStruct((H, D // 8, NBT, 8, CHUNK), jnp.float32),
    mesh=plsc.VectorSubcoreMesh(core_axis_name="c", subcore_axis_name="s"),
    scratch_types=[
        pltpu.VMEM((H, BT_PER_W * CHUNK), jnp.int32),
        pltpu.VMEM((2 * K, CHUNK, D), jnp.float32),
        pltpu.VMEM((TB, D, SKEW), jnp.float32),
        pltpu.SemaphoreType.DMA,
        pltpu.SemaphoreType.DMA,
        pltpu.SemaphoreType.DMA,
        pltpu.SemaphoreType.DMA,
    ],
    compiler_params=pltpu.CompilerParams(use_tc_tiling_on_sc=False,
                                         needs_layout_passes=False),
)
def _emb_lookup(table_hbm, ids_hbm, out_hbm, idsv_v, rows_v, tbuf_v,
                gsem_a, gsem_b, wsem, isem):
    _emb_body(table_hbm, ids_hbm, out_hbm, idsv_v, rows_v, tbuf_v,
              gsem_a, gsem_b, wsem, isem)


def kernel(input_ids, weight_shard):
    ids_t = input_ids.astype(jnp.int32).T
    out5 = _emb_lookup(weight_shard, ids_t)
    return out5.transpose(2, 4, 0, 1, 3).reshape(B, H, D)
